# baseline (speedup 1.0000x reference)
import jax
import jax.numpy as jnp
from jax import lax
from jax.experimental import pallas as pl
from jax.experimental.pallas import tpu as pltpu

N = 10000
D = 128
K = N // 2
BLK = 400


def _tc_topk_body(x_ref, w_ref, b_ref, idx_ref, s_ref):
    X = x_ref[...]
    Wv = w_ref[...]
    bb = b_ref[0, 0]
    s_ref[...] = jnp.sum(X * Wv, axis=1, keepdims=True) + bb
    dn = (((1,), (1,)), ((), ()))
    ones11 = jnp.ones((1, 1), jnp.float32)
    s_row = lax.dot_general(ones11, s_ref[...], dn,
                            preferred_element_type=jnp.float32)
    j_row = lax.broadcasted_iota(jnp.int32, (1, N), 1)
    p_row = lax.broadcasted_iota(jnp.int32, (1, K), 1).astype(jnp.float32)

    def blk(ib, acc):
        s_blk = s_ref[pl.ds(ib * BLK, BLK), :]
        i_blk = lax.broadcasted_iota(jnp.int32, (BLK, 1), 0) + ib * BLK
        gt = s_row > s_blk
        tie = (s_row == s_blk) & (j_row < i_blk)
        rank = jnp.sum(jnp.where(gt | tie, 1.0, 0.0), axis=1, keepdims=True)
        sel = rank == p_row
        contrib = jnp.sum(jnp.where(sel, i_blk.astype(jnp.float32), 0.0),
                          axis=0, keepdims=True)
        return acc + contrib

    acc = lax.fori_loop(0, N // BLK, blk, jnp.zeros((1, K), jnp.float32))
    idx_ref[...] = acc.astype(jnp.int32)


_tc_topk = pl.pallas_call(
    _tc_topk_body,
    out_shape=jax.ShapeDtypeStruct((1, K), jnp.int32),
    scratch_shapes=[pltpu.VMEM((N, 1), jnp.float32)],
)


def kernel(node_features, adjacency_matrix, W, b):
    idx = _tc_topk(node_features, W, b.reshape(1, 1)).reshape(K)
    feat = jnp.take(node_features, idx, axis=0)
    adj = jnp.take(jnp.take(adjacency_matrix, idx, axis=0), idx, axis=1)
    return (feat, adj)


# speedup vs baseline: 1.0004x; 1.0004x over previous
import jax
import jax.numpy as jnp
from jax import lax
from jax.experimental import pallas as pl
from jax.experimental.pallas import tpu as pltpu

N = 10000
D = 128
K = N // 2
BLK = 1000


def _tc_topk_body(x_ref, w_ref, b_ref, idx_ref, s_ref):
    X = x_ref[...]
    Wv = w_ref[...]
    bb = b_ref[0, 0]
    s_ref[...] = jnp.sum(X * Wv, axis=1, keepdims=True) + bb
    dn = (((1,), (1,)), ((), ()))
    ones11 = jnp.ones((1, 1), jnp.float32)
    s_row = lax.dot_general(ones11, s_ref[...], dn,
                            preferred_element_type=jnp.float32)
    j_row = lax.broadcasted_iota(jnp.int32, (1, N), 1)
    p_row = lax.broadcasted_iota(jnp.int32, (1, K), 1).astype(jnp.float32)

    def blk(ib, acc):
        s_blk = s_ref[pl.ds(ib * BLK, BLK), :]
        i_blk = lax.broadcasted_iota(jnp.int32, (BLK, 1), 0) + ib * BLK
        gt = s_row > s_blk
        tie = (s_row == s_blk) & (j_row < i_blk)
        rank = jnp.sum(jnp.where(gt | tie, 1.0, 0.0), axis=1, keepdims=True)
        sel = rank == p_row
        contrib = jnp.sum(jnp.where(sel, i_blk.astype(jnp.float32), 0.0),
                          axis=0, keepdims=True)
        return acc + contrib

    acc = lax.fori_loop(0, N // BLK, blk, jnp.zeros((1, K), jnp.float32))
    idx_ref[...] = acc.astype(jnp.int32)


_tc_topk = pl.pallas_call(
    _tc_topk_body,
    out_shape=jax.ShapeDtypeStruct((1, K), jnp.int32),
    scratch_shapes=[pltpu.VMEM((N, 1), jnp.float32)],
)


def kernel(node_features, adjacency_matrix, W, b):
    idx = _tc_topk(node_features, W, b.reshape(1, 1)).reshape(K)
    feat = jnp.take(node_features, idx, axis=0)
    adj = jnp.take(jnp.take(adjacency_matrix, idx, axis=0), idx, axis=1)
    return (feat, adj)
